# baseline (speedup 1.0000x reference)
import jax
import jax.numpy as jnp
from jax.experimental import pallas as pl
from jax.experimental.pallas import tpu as pltpu

_N = 4096
_D = 256
_K = 32
_BETA = 1.0
_EPS = 1e-5
_BM = 256
_NB = _N // _BM


def _main_body(f_full_ref, f_rows_ref, sqr_ref, sqc_ref, w_ref, b_ref,
               g2_ref, b2_ref, a_ref, h_ref, scr0_ref, scr1_ref):
    i = pl.program_id(0)
    f = f_full_ref[...]

    def compute_dist(scr_w):
        fi = f_rows_ref[...]
        sq_all = sqr_ref[...]
        sq_i = sqc_ref[...]
        cross = jax.lax.dot_general(
            fi, f, (((1,), (1,)), ((), ())),
            precision=jax.lax.Precision.DEFAULT,
            preferred_element_type=jnp.float32)
        dist = sq_i + sq_all - 2.0 * cross
        cols = jax.lax.broadcasted_iota(jnp.int32, (_BM, _N), 1)
        rows_g = (jnp.minimum(i, _NB - 1) * _BM
                  + jax.lax.broadcasted_iota(jnp.int32, (_BM, _N), 0))
        big_diag = jnp.float32(3.2e38)
        scr_w[...] = jnp.where(cols == rows_g, big_diag, dist)

    def select_and_out(scr_r):
        big_sel = jnp.float32(2.8e38)
        grp = _N // 8
        d = scr_r[...]
        dsl = [d[:, j * grp:(j + 1) * grp] for j in range(8)]
        s = list(dsl)

        def _ce(ii, jj):
            lo = jnp.minimum(s[ii], s[jj])
            hi = jnp.maximum(s[ii], s[jj])
            s[ii] = lo
            s[jj] = hi

        for (ii, jj) in ((0, 1), (2, 3), (4, 5), (6, 7),
                         (0, 2), (1, 3), (4, 6), (5, 7),
                         (1, 2), (5, 6),
                         (0, 4), (1, 5), (2, 6), (3, 7),
                         (2, 4), (3, 5),
                         (1, 2), (3, 4), (5, 6)):
            _ce(ii, jj)
        cols_q = jax.lax.broadcasted_iota(jnp.int32, (_BM, grp), 1)
        for _ in range(_K):
            amin = jnp.argmin(s[0], axis=1)[:, None]
            taken = cols_q == amin
            for j in range(7):
                s[j] = jnp.where(taken, s[j + 1], s[j])
            s[7] = jnp.where(taken, big_sel, s[7])
        inv_k = jnp.float32(1.0 / _K)
        zero = jnp.float32(0.0)
        for j in range(8):
            a_ref[:, j * grp:(j + 1) * grp] = jnp.where(
                dsl[j] < s[0], inv_k, zero)

        prop = jax.lax.dot_general(
            a_ref[...], f, (((1,), (0,)), ((), ())),
            preferred_element_type=jnp.float32)
        proj = jax.lax.dot_general(
            prop, w_ref[...], (((1,), (1,)), ((), ())),
            preferred_element_type=jnp.float32) + b_ref[...]
        mu = jnp.mean(proj, axis=-1, keepdims=True)
        var = jnp.mean((proj - mu) ** 2, axis=-1, keepdims=True)
        h_ref[...] = ((proj - mu) / jnp.sqrt(var + _EPS) * g2_ref[...]
                      + b2_ref[...])

    @pl.when(i % 2 == 0)
    def _():
        compute_dist(scr0_ref)
        select_and_out(scr1_ref)

    @pl.when(i % 2 == 1)
    def _():
        compute_dist(scr1_ref)
        select_and_out(scr0_ref)


def kernel(Z, Y, ln1_g, ln1_b, W, b, ln2_g, ln2_b):
    x = Z + _BETA * Y
    mu = jnp.mean(x, axis=-1, keepdims=True)
    var = jnp.mean((x - mu) ** 2, axis=-1, keepdims=True)
    fused = (x - mu) / jnp.sqrt(var + _EPS) * ln1_g + ln1_b

    sq = jnp.sum(fused * fused, axis=1)

    def _in_rows(i):
        return (jnp.minimum(i, _NB - 1), 0)

    def _out_rows(i):
        return (jnp.maximum(i - 1, 0), 0)

    a, hidden = pl.pallas_call(
        _main_body,
        grid=(_NB + 1,),
        in_specs=[
            pl.BlockSpec((_N, _D), lambda i: (0, 0)),
            pl.BlockSpec((_BM, _D), _in_rows),
            pl.BlockSpec((1, _N), lambda i: (0, 0)),
            pl.BlockSpec((_BM, 1), _in_rows),
            pl.BlockSpec((_D, _D), lambda i: (0, 0)),
            pl.BlockSpec((1, _D), lambda i: (0, 0)),
            pl.BlockSpec((1, _D), lambda i: (0, 0)),
            pl.BlockSpec((1, _D), lambda i: (0, 0)),
        ],
        out_specs=[
            pl.BlockSpec((_BM, _N), _out_rows),
            pl.BlockSpec((_BM, _D), _out_rows),
        ],
        out_shape=[
            jax.ShapeDtypeStruct((_N, _N), jnp.float32),
            jax.ShapeDtypeStruct((_N, _D), jnp.float32),
        ],
        scratch_shapes=[
            pltpu.VMEM((_BM, _N), jnp.float32),
            pltpu.VMEM((_BM, _N), jnp.float32),
        ],
    )(fused, fused, sq.reshape(1, _N), sq.reshape(_N, 1), W,
      b.reshape(1, _D), ln2_g.reshape(1, _D), ln2_b.reshape(1, _D))

    return fused, a, hidden


# speedup vs baseline: 1.1776x; 1.1776x over previous
import jax
import jax.numpy as jnp
from jax.experimental import pallas as pl

_N = 4096
_D = 256
_K = 32
_BETA = 1.0
_EPS = 1e-5
_BM = 256


def _main_body(f_full_ref, f_rows_ref, sqr_ref, sqc_ref, w_ref, b_ref,
               g2_ref, b2_ref, a_ref, h_ref):
    i = pl.program_id(0)
    f = f_full_ref[...]
    fi = f_rows_ref[...]

    sq_all = sqr_ref[...]
    sq_i = sqc_ref[...]

    cross = jax.lax.dot_general(
        fi, f, (((1,), (1,)), ((), ())),
        precision=jax.lax.Precision.DEFAULT,
        preferred_element_type=jnp.float32)
    dist = sq_i + sq_all - 2.0 * cross

    cols = jax.lax.broadcasted_iota(jnp.int32, (_BM, _N), 1)
    rows_g = i * _BM + jax.lax.broadcasted_iota(jnp.int32, (_BM, _N), 0)
    big_diag = jnp.float32(3.2e38)
    big_sel = jnp.float32(2.8e38)
    d = jnp.where(cols == rows_g, big_diag, dist)

    ngrp = 16
    grp = _N // ngrp
    dsl = [d[:, j * grp:(j + 1) * grp] for j in range(ngrp)]
    s = list(dsl)

    def _ce(i, j):
        lo = jnp.minimum(s[i], s[j])
        hi = jnp.maximum(s[i], s[j])
        s[i] = lo
        s[j] = hi

    p = 1
    while p < ngrp:
        k = p
        while k >= 1:
            for j0 in range(k % p, ngrp - k, 2 * k):
                for i0 in range(0, min(k, ngrp - j0 - k)):
                    if (i0 + j0) // (p * 2) == (i0 + j0 + k) // (p * 2):
                        _ce(i0 + j0, i0 + j0 + k)
            k //= 2
        p *= 2
    cols_q = cols[:, :grp]
    for _ in range(_K):
        amin = jnp.argmin(s[0], axis=1)[:, None]
        taken = cols_q == amin
        for j in range(ngrp - 1):
            s[j] = jnp.where(taken, s[j + 1], s[j])
        s[ngrp - 1] = jnp.where(taken, big_sel, s[ngrp - 1])
    inv_k = jnp.float32(1.0 / _K)
    zero = jnp.float32(0.0)
    for j in range(ngrp):
        a_ref[:, j * grp:(j + 1) * grp] = jnp.where(
            dsl[j] < s[0], inv_k, zero)

    prop = jax.lax.dot_general(
        a_ref[...], f, (((1,), (0,)), ((), ())),
        preferred_element_type=jnp.float32)
    proj = jax.lax.dot_general(
        prop, w_ref[...], (((1,), (1,)), ((), ())),
        preferred_element_type=jnp.float32) + b_ref[...]
    mu = jnp.mean(proj, axis=-1, keepdims=True)
    var = jnp.mean((proj - mu) ** 2, axis=-1, keepdims=True)
    h_ref[...] = (proj - mu) / jnp.sqrt(var + _EPS) * g2_ref[...] + b2_ref[...]


def kernel(Z, Y, ln1_g, ln1_b, W, b, ln2_g, ln2_b):
    x = Z + _BETA * Y
    mu = jnp.mean(x, axis=-1, keepdims=True)
    var = jnp.mean((x - mu) ** 2, axis=-1, keepdims=True)
    fused = (x - mu) / jnp.sqrt(var + _EPS) * ln1_g + ln1_b

    sq = jnp.sum(fused * fused, axis=1)
    a, hidden = pl.pallas_call(
        _main_body,
        grid=(_N // _BM,),
        in_specs=[
            pl.BlockSpec((_N, _D), lambda i: (0, 0)),
            pl.BlockSpec((_BM, _D), lambda i: (i, 0)),
            pl.BlockSpec((1, _N), lambda i: (0, 0)),
            pl.BlockSpec((_BM, 1), lambda i: (i, 0)),
            pl.BlockSpec((_D, _D), lambda i: (0, 0)),
            pl.BlockSpec((1, _D), lambda i: (0, 0)),
            pl.BlockSpec((1, _D), lambda i: (0, 0)),
            pl.BlockSpec((1, _D), lambda i: (0, 0)),
        ],
        out_specs=[
            pl.BlockSpec((_BM, _N), lambda i: (i, 0)),
            pl.BlockSpec((_BM, _D), lambda i: (i, 0)),
        ],
        out_shape=[
            jax.ShapeDtypeStruct((_N, _N), jnp.float32),
            jax.ShapeDtypeStruct((_N, _D), jnp.float32),
        ],
    )(fused, fused, sq.reshape(1, _N), sq.reshape(_N, 1), W,
      b.reshape(1, _D), ln2_g.reshape(1, _D), ln2_b.reshape(1, _D))

    return fused, a, hidden


# speedup vs baseline: 1.5273x; 1.2970x over previous
import jax
import jax.numpy as jnp
from jax.experimental import pallas as pl

_N = 4096
_D = 256
_K = 32
_BETA = 1.0
_EPS = 1e-5
_BM = 256


def _main_body(f_full_ref, f_rows_ref, sqr_ref, sqc_ref, w_ref, b_ref,
               g2_ref, b2_ref, a_ref, h_ref):
    i = pl.program_id(0)
    f = f_full_ref[...]
    fi = f_rows_ref[...]

    sq_all = sqr_ref[...]
    sq_i = sqc_ref[...]

    cross = jax.lax.dot_general(
        fi, f, (((1,), (1,)), ((), ())),
        precision=jax.lax.Precision.DEFAULT,
        preferred_element_type=jnp.float32)
    dist = sq_i + sq_all - 2.0 * cross

    cols = jax.lax.broadcasted_iota(jnp.int32, (_BM, _N), 1)
    rows_g = i * _BM + jax.lax.broadcasted_iota(jnp.int32, (_BM, _N), 0)
    big_diag = jnp.float32(3.2e38)
    big_sel = jnp.float32(2.8e38)
    d = jnp.where(cols == rows_g, big_diag, dist)

    ngrp = 32
    grp = _N // ngrp
    dsl = [d[:, j * grp:(j + 1) * grp] for j in range(ngrp)]
    s = list(dsl)

    def _ce(i, j):
        lo = jnp.minimum(s[i], s[j])
        hi = jnp.maximum(s[i], s[j])
        s[i] = lo
        s[j] = hi

    p = 1
    while p < ngrp:
        k = p
        while k >= 1:
            for j0 in range(k % p, ngrp - k, 2 * k):
                for i0 in range(0, min(k, ngrp - j0 - k)):
                    if (i0 + j0) // (p * 2) == (i0 + j0 + k) // (p * 2):
                        _ce(i0 + j0, i0 + j0 + k)
            k //= 2
        p *= 2
    cols_q = cols[:, :grp]
    for _ in range(_K):
        amin = jnp.argmin(s[0], axis=1)[:, None]
        taken = cols_q == amin
        for j in range(ngrp - 1):
            s[j] = jnp.where(taken, s[j + 1], s[j])
        s[ngrp - 1] = jnp.where(taken, big_sel, s[ngrp - 1])
    inv_k = jnp.float32(1.0 / _K)
    zero = jnp.float32(0.0)
    for j in range(ngrp):
        a_ref[:, j * grp:(j + 1) * grp] = jnp.where(
            dsl[j] < s[0], inv_k, zero)

    prop = jax.lax.dot_general(
        a_ref[...], f, (((1,), (0,)), ((), ())),
        preferred_element_type=jnp.float32)
    proj = jax.lax.dot_general(
        prop, w_ref[...], (((1,), (1,)), ((), ())),
        preferred_element_type=jnp.float32) + b_ref[...]
    mu = jnp.mean(proj, axis=-1, keepdims=True)
    var = jnp.mean((proj - mu) ** 2, axis=-1, keepdims=True)
    h_ref[...] = (proj - mu) / jnp.sqrt(var + _EPS) * g2_ref[...] + b2_ref[...]


def kernel(Z, Y, ln1_g, ln1_b, W, b, ln2_g, ln2_b):
    x = Z + _BETA * Y
    mu = jnp.mean(x, axis=-1, keepdims=True)
    var = jnp.mean((x - mu) ** 2, axis=-1, keepdims=True)
    fused = (x - mu) / jnp.sqrt(var + _EPS) * ln1_g + ln1_b

    sq = jnp.sum(fused * fused, axis=1)
    a, hidden = pl.pallas_call(
        _main_body,
        grid=(_N // _BM,),
        in_specs=[
            pl.BlockSpec((_N, _D), lambda i: (0, 0)),
            pl.BlockSpec((_BM, _D), lambda i: (i, 0)),
            pl.BlockSpec((1, _N), lambda i: (0, 0)),
            pl.BlockSpec((_BM, 1), lambda i: (i, 0)),
            pl.BlockSpec((_D, _D), lambda i: (0, 0)),
            pl.BlockSpec((1, _D), lambda i: (0, 0)),
            pl.BlockSpec((1, _D), lambda i: (0, 0)),
            pl.BlockSpec((1, _D), lambda i: (0, 0)),
        ],
        out_specs=[
            pl.BlockSpec((_BM, _N), lambda i: (i, 0)),
            pl.BlockSpec((_BM, _D), lambda i: (i, 0)),
        ],
        out_shape=[
            jax.ShapeDtypeStruct((_N, _N), jnp.float32),
            jax.ShapeDtypeStruct((_N, _D), jnp.float32),
        ],
    )(fused, fused, sq.reshape(1, _N), sq.reshape(_N, 1), W,
      b.reshape(1, _D), ln2_g.reshape(1, _D), ln2_b.reshape(1, _D))

    return fused, a, hidden


# speedup vs baseline: 1.6004x; 1.0478x over previous
import jax
import jax.numpy as jnp
from jax.experimental import pallas as pl

_N = 4096
_D = 256
_K = 32
_BETA = 1.0
_EPS = 1e-5
_BM = 512


def _main_body(f_full_ref, f_rows_ref, sqr_ref, sqc_ref, w_ref, b_ref,
               g2_ref, b2_ref, a_ref, h_ref):
    i = pl.program_id(0)
    f = f_full_ref[...]
    fi = f_rows_ref[...]

    sq_all = sqr_ref[...]
    sq_i = sqc_ref[...]

    cross = jax.lax.dot_general(
        fi, f, (((1,), (1,)), ((), ())),
        precision=jax.lax.Precision.DEFAULT,
        preferred_element_type=jnp.float32)
    dist = sq_i + sq_all - 2.0 * cross

    cols = jax.lax.broadcasted_iota(jnp.int32, (_BM, _N), 1)
    rows_g = i * _BM + jax.lax.broadcasted_iota(jnp.int32, (_BM, _N), 0)
    big_diag = jnp.float32(3.2e38)
    big_sel = jnp.float32(2.8e38)
    d = jnp.where(cols == rows_g, big_diag, dist)

    ngrp = 32
    grp = _N // ngrp
    dsl = [d[:, j * grp:(j + 1) * grp] for j in range(ngrp)]
    s = list(dsl)

    def _ce(i, j):
        lo = jnp.minimum(s[i], s[j])
        hi = jnp.maximum(s[i], s[j])
        s[i] = lo
        s[j] = hi

    p = 1
    while p < ngrp:
        k = p
        while k >= 1:
            for j0 in range(k % p, ngrp - k, 2 * k):
                for i0 in range(0, min(k, ngrp - j0 - k)):
                    if (i0 + j0) // (p * 2) == (i0 + j0 + k) // (p * 2):
                        _ce(i0 + j0, i0 + j0 + k)
            k //= 2
        p *= 2
    cols_q = cols[:, :grp]
    for _ in range(_K):
        amin = jnp.argmin(s[0], axis=1)[:, None]
        taken = cols_q == amin
        for j in range(ngrp - 1):
            s[j] = jnp.where(taken, s[j + 1], s[j])
        s[ngrp - 1] = jnp.where(taken, big_sel, s[ngrp - 1])
    inv_k = jnp.float32(1.0 / _K)
    zero = jnp.float32(0.0)
    for j in range(ngrp):
        a_ref[:, j * grp:(j + 1) * grp] = jnp.where(
            dsl[j] < s[0], inv_k, zero)

    prop = jax.lax.dot_general(
        a_ref[...], f, (((1,), (0,)), ((), ())),
        preferred_element_type=jnp.float32)
    proj = jax.lax.dot_general(
        prop, w_ref[...], (((1,), (1,)), ((), ())),
        preferred_element_type=jnp.float32) + b_ref[...]
    mu = jnp.mean(proj, axis=-1, keepdims=True)
    var = jnp.mean((proj - mu) ** 2, axis=-1, keepdims=True)
    h_ref[...] = (proj - mu) / jnp.sqrt(var + _EPS) * g2_ref[...] + b2_ref[...]


def kernel(Z, Y, ln1_g, ln1_b, W, b, ln2_g, ln2_b):
    x = Z + _BETA * Y
    mu = jnp.mean(x, axis=-1, keepdims=True)
    var = jnp.mean((x - mu) ** 2, axis=-1, keepdims=True)
    fused = (x - mu) / jnp.sqrt(var + _EPS) * ln1_g + ln1_b

    sq = jnp.sum(fused * fused, axis=1)
    a, hidden = pl.pallas_call(
        _main_body,
        grid=(_N // _BM,),
        in_specs=[
            pl.BlockSpec((_N, _D), lambda i: (0, 0)),
            pl.BlockSpec((_BM, _D), lambda i: (i, 0)),
            pl.BlockSpec((1, _N), lambda i: (0, 0)),
            pl.BlockSpec((_BM, 1), lambda i: (i, 0)),
            pl.BlockSpec((_D, _D), lambda i: (0, 0)),
            pl.BlockSpec((1, _D), lambda i: (0, 0)),
            pl.BlockSpec((1, _D), lambda i: (0, 0)),
            pl.BlockSpec((1, _D), lambda i: (0, 0)),
        ],
        out_specs=[
            pl.BlockSpec((_BM, _N), lambda i: (i, 0)),
            pl.BlockSpec((_BM, _D), lambda i: (i, 0)),
        ],
        out_shape=[
            jax.ShapeDtypeStruct((_N, _N), jnp.float32),
            jax.ShapeDtypeStruct((_N, _D), jnp.float32),
        ],
    )(fused, fused, sq.reshape(1, _N), sq.reshape(_N, 1), W,
      b.reshape(1, _D), ln2_g.reshape(1, _D), ln2_b.reshape(1, _D))

    return fused, a, hidden
